# baseline (speedup 1.0000x reference)
import functools

import jax
import jax.numpy as jnp
from jax import lax
from jax.experimental import pallas as pl
from jax.experimental.pallas import tpu as pltpu
from jax.experimental.pallas import tpu_sc as plsc

B, CH, H, W = 4, 3, 512, 512
HW = H * W
WG = W // 4
P = 256
NW = 32
CPT = B * HW // P // NW
NSEG = 8 * P // 128
GROUPS = P // 16


def _floor_i(v):
    t = v.astype(jnp.int32)
    return jnp.where(t.astype(jnp.float32) > v, t - 1, t)


def _sc_body(img_hbm, flow_hbm, g16_hbm,
             fx_v, fy_v, o0_v, idx_v, w_v, g_v, ob_v, flush_s, sem):
    sid = lax.axis_index("s")
    wid = sid * 2 + lax.axis_index("c")
    lanes = lax.iota(jnp.int32, 16)

    def chunk_body(t, carry):
        cid = wid * CPT + t
        b = cid >> 10
        off = (cid & 1023) * P
        pltpu.sync_copy(flow_hbm.at[b, 0, pl.ds(off, P)], fx_v)
        pltpu.sync_copy(flow_hbm.at[b, 1, pl.ds(off, P)], fy_v)
        base4 = b * (HW // 4)

        def grp_a(i, c2):
            p0 = i * 16
            pix = off + p0 + lanes
            xi = pix & (W - 1)
            yi = pix >> 9
            fx = fx_v[pl.ds(p0, 16)]
            fy = fy_v[pl.ds(p0, 16)]
            bx = _floor_i(xi.astype(jnp.float32) + fx)
            by = _floor_i(yi.astype(jnp.float32) + fy)
            tu = fx - _floor_i(fx).astype(jnp.float32)
            tv = fy - _floor_i(fy).astype(jnp.float32)
            u0 = 1.0 - tu
            v0 = 1.0 - tv
            xs = bx - 1
            m0 = xs >> 2
            o0_v[pl.ds(p0, 16)] = xs - (m0 << 2)
            m0c = jnp.minimum(jnp.maximum(m0, 0), WG - 1)
            m1c = jnp.minimum(jnp.maximum(m0 + 1, 0), WG - 1)
            vx = []
            for j in range(4):
                xk = xs + j
                vx.append((xk >= 0) & (xk < W))
            wxy = [u0 * v0, u0 * tv, tu * v0, tu * tv]
            prow = p0 >> 7
            pcol = p0 & 127
            for d in range(4):
                yk = by + (d - 1)
                vy = (yk >= 0) & (yk < H)
                ys = jnp.where(vy, yk, 0)
                basey = base4 + (ys << 7)
                idx_v[2 * (2 * d) + prow, pl.ds(pcol, 16)] = basey + m0c
                idx_v[2 * (2 * d + 1) + prow, pl.ds(pcol, 16)] = basey + m1c
                for j in range(4):
                    k = 4 * j + d
                    wk = wxy[(j // 2) * 2 + (d // 2)]
                    w_v[k, pl.ds(p0, 16)] = jnp.where(vy & vx[j], wk, 0.0)
            return c2

        lax.fori_loop(0, GROUPS, grp_a, 0)
        pltpu.sync_copy(idx_v, flush_s.at[sid])
        copies = [
            pltpu.async_copy(img_hbm.at[idx_v.at[s]],
                             g_v.at[pl.ds(s * 128, 128)], sem)
            for s in range(NSEG)
        ]
        for cp in copies:
            cp.wait()

        def grp_b(i, c2):
            p0 = i * 16
            o0 = o0_v[pl.ds(p0, 16)]
            pv = p0 + lanes
            for d in range(4):
                for j in range(4):
                    k = 4 * j + d
                    wv = w_v[k, pl.ds(p0, 16)]
                    oj = o0 + j
                    rowv = (2 * d) * P + ((oj >> 2) << 8) + pv
                    colb = (oj & 3) << 2
                    for c in range(CH):
                        gv = plsc.load_gather(g_v, [rowv, colb + c])
                        ob_v[k * CH + c, pl.ds(p0, 16)] = gv * wv
            return c2

        lax.fori_loop(0, GROUPS, grp_b, 0)
        pltpu.sync_copy(ob_v, g16_hbm.at[b, :, pl.ds(off, P)])
        return carry

    lax.fori_loop(0, CPT, chunk_body, 0)


def _tc_body(g_ref, k_ref, o_ref):
    acc = jnp.zeros((32, 128), jnp.float32)
    for k in range(16):
        kb = k_ref[0, k].reshape(32, 128)
        acc = acc + g_ref[0, k, 0] * kb
    o_ref[0, 0] = acc


def kernel(image, kernel, flow):
    img4 = jnp.pad(jnp.transpose(image, (0, 2, 3, 1)),
                   ((0, 0), (0, 0), (0, 0), (0, 1))).reshape(B * HW // 4, 16)
    fl = flow.reshape(B, 2, HW)
    mesh = plsc.VectorSubcoreMesh(core_axis_name="c", subcore_axis_name="s",
                                  num_cores=2, num_subcores=16)
    g16 = pl.kernel(
        _sc_body,
        out_type=jax.ShapeDtypeStruct((B, 16 * CH, HW), jnp.float32),
        mesh=mesh,
        compiler_params=pltpu.CompilerParams(needs_layout_passes=False,
                                             use_tc_tiling_on_sc=False),
        scratch_types=[
            pltpu.VMEM((P,), jnp.float32),
            pltpu.VMEM((P,), jnp.float32),
            pltpu.VMEM((P,), jnp.int32),
            pltpu.VMEM((NSEG, 128), jnp.int32),
            pltpu.VMEM((16, P), jnp.float32),
            pltpu.VMEM((8 * P, 16), jnp.float32),
            pltpu.VMEM((16 * CH, P), jnp.float32),
            pltpu.VMEM_SHARED((16, NSEG, 128), jnp.int32),
            pltpu.SemaphoreType.DMA,
        ],
    )(img4, fl)
    g16r = g16.reshape(B, 16, CH, HW // 128, 128)
    out = pl.pallas_call(
        _tc_body,
        out_shape=jax.ShapeDtypeStruct((B, CH, HW // 128, 128), jnp.float32),
        grid=(B, CH, HW // (128 * 32)),
        in_specs=[
            pl.BlockSpec((1, 16, 1, 32, 128),
                         lambda b, c, i: (b, 0, c, i, 0)),
            pl.BlockSpec((1, 16, 8, W),
                         lambda b, c, i: (b, 0, i, 0)),
        ],
        out_specs=pl.BlockSpec((1, 1, 32, 128), lambda b, c, i: (b, c, i, 0)),
    )(g16r, kernel)
    return out.reshape(B, CH, H, W)


# speedup vs baseline: 1.0008x; 1.0008x over previous
import functools

import jax
import jax.numpy as jnp
from jax import lax
from jax.experimental import pallas as pl
from jax.experimental.pallas import tpu as pltpu
from jax.experimental.pallas import tpu_sc as plsc

B, CH, H, W = 4, 3, 512, 512
HW = H * W
WG = W // 4
P = 256
NW = 32
CPT = B * HW // P // NW
NSEG = 8 * P // 128
GROUPS = P // 16


def _floor_i(v):
    t = v.astype(jnp.int32)
    return jnp.where(t.astype(jnp.float32) > v, t - 1, t)


def _sc_body(img_hbm, flow_hbm, g16_hbm,
             fx_v, fy_v, o0_v, idx_v, w_v, g_v, ob_v, flush_s, sem):
    sid = lax.axis_index("s")
    wid = sid * 2 + lax.axis_index("c")
    lanes = lax.iota(jnp.int32, 16)

    def chunk_body(t, carry):
        cid = wid * CPT + t
        b = cid >> 10
        off = (cid & 1023) * P
        pltpu.sync_copy(flow_hbm.at[b, 0, pl.ds(off, P)], fx_v)
        pltpu.sync_copy(flow_hbm.at[b, 1, pl.ds(off, P)], fy_v)
        base4 = b * (HW // 4)

        def grp_a(i, c2):
            p0 = i * 16
            pix = off + p0 + lanes
            xi = pix & (W - 1)
            yi = pix >> 9
            fx = fx_v[pl.ds(p0, 16)]
            fy = fy_v[pl.ds(p0, 16)]
            bx = _floor_i(xi.astype(jnp.float32) + fx)
            by = _floor_i(yi.astype(jnp.float32) + fy)
            tu = fx - _floor_i(fx).astype(jnp.float32)
            tv = fy - _floor_i(fy).astype(jnp.float32)
            u0 = 1.0 - tu
            v0 = 1.0 - tv
            xs = bx - 1
            m0 = xs >> 2
            o0_v[pl.ds(p0, 16)] = xs - (m0 << 2)
            m0c = jnp.minimum(jnp.maximum(m0, 0), WG - 1)
            m1c = jnp.minimum(jnp.maximum(m0 + 1, 0), WG - 1)
            vx = []
            for j in range(4):
                xk = xs + j
                vx.append((xk >= 0) & (xk < W))
            wxy = [u0 * v0, u0 * tv, tu * v0, tu * tv]
            prow = p0 >> 7
            pcol = p0 & 127
            for d in range(4):
                yk = by + (d - 1)
                vy = (yk >= 0) & (yk < H)
                ys = jnp.where(vy, yk, 0)
                basey = base4 + (ys << 7)
                idx_v[2 * (2 * d) + prow, pl.ds(pcol, 16)] = basey + m0c
                idx_v[2 * (2 * d + 1) + prow, pl.ds(pcol, 16)] = basey + m1c
                for j in range(4):
                    k = 4 * j + d
                    wk = wxy[(j // 2) * 2 + (d // 2)]
                    w_v[k, pl.ds(p0, 16)] = jnp.where(vy & vx[j], wk, 0.0)
            return c2

        lax.fori_loop(0, GROUPS, grp_a, 0)
        pltpu.sync_copy(idx_v, flush_s.at[sid])
        copies = [
            pltpu.async_copy(img_hbm.at[idx_v.at[s]],
                             g_v.at[pl.ds(s * 128, 128)], sem)
            for s in range(NSEG)
        ]
        for cp in copies:
            cp.wait()

        def grp_b(i, c2):
            p0 = i * 16
            o0 = o0_v[pl.ds(p0, 16)]
            pv = p0 + lanes
            for d in range(4):
                for j in range(4):
                    k = 4 * j + d
                    wv = w_v[k, pl.ds(p0, 16)]
                    oj = o0 + j
                    rowv = (2 * d) * P + ((oj >> 2) << 8) + pv
                    colb = (oj & 3) << 2
                    for c in range(CH):
                        gv = plsc.load_gather(g_v, [rowv, colb + c])
                        ob_v[k * CH + c, p0 >> 7, pl.ds(p0 & 127, 16)] = gv * wv
            return c2

        lax.fori_loop(0, GROUPS, grp_b, 0)
        pltpu.sync_copy(ob_v, g16_hbm.at[b, :, pl.ds(off >> 7, P // 128), :])
        return carry

    lax.fori_loop(0, CPT, chunk_body, 0)


def _tc_body(g_ref, k_ref, o_ref):
    acc = jnp.zeros((32, 128), jnp.float32)
    for k in range(16):
        kb = k_ref[0, k].reshape(32, 128)
        acc = acc + g_ref[0, k, 0] * kb
    o_ref[0, 0] = acc


def kernel(image, kernel, flow):
    img4 = jnp.pad(jnp.transpose(image, (0, 2, 3, 1)),
                   ((0, 0), (0, 0), (0, 0), (0, 1))).reshape(B * HW // 4, 16)
    fl = flow.reshape(B, 2, HW)
    mesh = plsc.VectorSubcoreMesh(core_axis_name="c", subcore_axis_name="s",
                                  num_cores=2, num_subcores=16)
    g16 = pl.kernel(
        _sc_body,
        out_type=jax.ShapeDtypeStruct((B, 16 * CH, HW // 128, 128),
                                      jnp.float32),
        mesh=mesh,
        compiler_params=pltpu.CompilerParams(needs_layout_passes=False,
                                             use_tc_tiling_on_sc=False),
        scratch_types=[
            pltpu.VMEM((P,), jnp.float32),
            pltpu.VMEM((P,), jnp.float32),
            pltpu.VMEM((P,), jnp.int32),
            pltpu.VMEM((NSEG, 128), jnp.int32),
            pltpu.VMEM((16, P), jnp.float32),
            pltpu.VMEM((8 * P, 16), jnp.float32),
            pltpu.VMEM((16 * CH, P // 128, 128), jnp.float32),
            pltpu.VMEM_SHARED((16, NSEG, 128), jnp.int32),
            pltpu.SemaphoreType.DMA,
        ],
    )(img4, fl)
    g16r = g16.reshape(B, 16, CH, HW // 128, 128)
    out = pl.pallas_call(
        _tc_body,
        out_shape=jax.ShapeDtypeStruct((B, CH, HW // 128, 128), jnp.float32),
        grid=(B, CH, HW // (128 * 32)),
        in_specs=[
            pl.BlockSpec((1, 16, 1, 32, 128),
                         lambda b, c, i: (b, 0, c, i, 0)),
            pl.BlockSpec((1, 16, 8, W),
                         lambda b, c, i: (b, 0, i, 0)),
        ],
        out_specs=pl.BlockSpec((1, 1, 32, 128), lambda b, c, i: (b, c, i, 0)),
    )(g16r, kernel)
    return out.reshape(B, CH, H, W)


# speedup vs baseline: 1.3178x; 1.3167x over previous
import functools

import jax
import jax.numpy as jnp
from jax import lax
from jax.experimental import pallas as pl
from jax.experimental.pallas import tpu as pltpu
from jax.experimental.pallas import tpu_sc as plsc

B, CH, H, W = 4, 3, 512, 512
HW = H * W
WG = W // 4
P = 256
NW = 32
CPT = B * HW // P // NW
NSEG = 8 * P // 128
GROUPS = P // 16


def _floor_i(v):
    t = v.astype(jnp.int32)
    return jnp.where(t.astype(jnp.float32) > v, t - 1, t)


def _warp_body(img_hbm, kern_hbm, flow_hbm, out_hbm,
               fx_v, fy_v, kb_v, o0_v, idx_v, w_v, g_v, ob_v, flush_s, sem):
    sid = lax.axis_index("s")
    wid = sid * 2 + lax.axis_index("c")
    lanes = lax.iota(jnp.int32, 16)

    def chunk_body(t, carry):
        cid = wid * CPT + t
        b = cid >> 10
        off = (cid & 1023) * P
        pltpu.sync_copy(flow_hbm.at[b, 0, pl.ds(off, P)], fx_v)
        pltpu.sync_copy(flow_hbm.at[b, 1, pl.ds(off, P)], fy_v)
        pltpu.sync_copy(kern_hbm.at[b, :, pl.ds(off, P)], kb_v)
        base4 = b * (HW // 4)

        def grp_a(i, c2):
            p0 = i * 16
            pix = off + p0 + lanes
            xi = pix & (W - 1)
            yi = pix >> 9
            fx = fx_v[pl.ds(p0, 16)]
            fy = fy_v[pl.ds(p0, 16)]
            bx = _floor_i(xi.astype(jnp.float32) + fx)
            by = _floor_i(yi.astype(jnp.float32) + fy)
            tu = fx - _floor_i(fx).astype(jnp.float32)
            tv = fy - _floor_i(fy).astype(jnp.float32)
            u0 = 1.0 - tu
            v0 = 1.0 - tv
            xs = bx - 1
            m0 = xs >> 2
            o0_v[pl.ds(p0, 16)] = xs - (m0 << 2)
            m0c = jnp.minimum(jnp.maximum(m0, 0), WG - 1)
            m1c = jnp.minimum(jnp.maximum(m0 + 1, 0), WG - 1)
            vx = []
            for j in range(4):
                xk = xs + j
                vx.append((xk >= 0) & (xk < W))
            wxy = [u0 * v0, u0 * tv, tu * v0, tu * tv]
            prow = p0 >> 7
            pcol = p0 & 127
            for d in range(4):
                yk = by + (d - 1)
                vy = (yk >= 0) & (yk < H)
                ys = jnp.where(vy, yk, 0)
                basey = base4 + (ys << 7)
                idx_v[2 * (2 * d) + prow, pl.ds(pcol, 16)] = basey + m0c
                idx_v[2 * (2 * d + 1) + prow, pl.ds(pcol, 16)] = basey + m1c
                for j in range(4):
                    k = 4 * j + d
                    wk = kb_v[k, pl.ds(p0, 16)] * wxy[(j // 2) * 2 + (d // 2)]
                    w_v[k, pl.ds(p0, 16)] = jnp.where(vy & vx[j], wk, 0.0)
            return c2

        lax.fori_loop(0, GROUPS, grp_a, 0)
        pltpu.sync_copy(idx_v, flush_s.at[sid])
        copies = [
            pltpu.async_copy(img_hbm.at[idx_v.at[s]],
                             g_v.at[pl.ds(s * 128, 128)], sem)
            for s in range(NSEG)
        ]
        for cp in copies:
            cp.wait()

        def grp_b(i, c2):
            p0 = i * 16
            acc = [jnp.zeros((16,), jnp.float32) for _ in range(CH)]
            o0 = o0_v[pl.ds(p0, 16)]
            pv = p0 + lanes
            for d in range(4):
                for j in range(4):
                    k = 4 * j + d
                    wv = w_v[k, pl.ds(p0, 16)]
                    oj = o0 + j
                    rowv = (2 * d) * P + ((oj >> 2) << 8) + pv
                    colb = (oj & 3) << 2
                    for c in range(CH):
                        gv = plsc.load_gather(g_v, [rowv, colb + c])
                        acc[c] = acc[c] + gv * wv
            for c in range(CH):
                ob_v[c, pl.ds(p0, 16)] = acc[c]
            return c2

        lax.fori_loop(0, GROUPS, grp_b, 0)
        pltpu.sync_copy(ob_v, out_hbm.at[b, :, pl.ds(off, P)])
        return carry

    lax.fori_loop(0, CPT, chunk_body, 0)


def kernel(image, kernel, flow):
    img4 = jnp.pad(jnp.transpose(image, (0, 2, 3, 1)),
                   ((0, 0), (0, 0), (0, 0), (0, 1))).reshape(B * HW // 4, 16)
    kern = kernel.reshape(B, 16, HW)
    fl = flow.reshape(B, 2, HW)
    mesh = plsc.VectorSubcoreMesh(core_axis_name="c", subcore_axis_name="s",
                                  num_cores=2, num_subcores=16)
    out = pl.kernel(
        _warp_body,
        out_type=jax.ShapeDtypeStruct((B, CH, HW), jnp.float32),
        mesh=mesh,
        compiler_params=pltpu.CompilerParams(needs_layout_passes=False,
                                             use_tc_tiling_on_sc=False),
        scratch_types=[
            pltpu.VMEM((P,), jnp.float32),
            pltpu.VMEM((P,), jnp.float32),
            pltpu.VMEM((16, P), jnp.float32),
            pltpu.VMEM((P,), jnp.int32),
            pltpu.VMEM((NSEG, 128), jnp.int32),
            pltpu.VMEM((16, P), jnp.float32),
            pltpu.VMEM((8 * P, 16), jnp.float32),
            pltpu.VMEM((CH, P), jnp.float32),
            pltpu.VMEM_SHARED((16, NSEG, 128), jnp.int32),
            pltpu.SemaphoreType.DMA,
        ],
    )(img4, kern, fl)
    return out.reshape(B, CH, H, W)
